# baseline (speedup 1.0000x reference)
import functools

import jax
import jax.numpy as jnp
from jax import lax
from jax.experimental import pallas as pl
from jax.experimental.pallas import tpu as pltpu
from jax.experimental.pallas import tpu_sc as plsc

NC = 2
NS = 16
LANES = 16
CHUNK = 128
SC_ROWS = 4096
BLK = 1024


def _make_sc_segment_sum(d, b, sc_rows):
    nw = NC * NS
    rows_per_w = sc_rows // nw
    m = rows_per_w // CHUNK
    mesh = plsc.VectorSubcoreMesh(
        core_axis_name="c", subcore_axis_name="s",
        num_cores=NC, num_subcores=NS)

    @functools.partial(
        pl.kernel,
        out_type=(
            jax.ShapeDtypeStruct((NC, 2, b, d), jnp.float32),
            jax.ShapeDtypeStruct((NC, 2, b, LANES), jnp.float32),
        ),
        mesh=mesh,
        scratch_types=[
            pltpu.VMEM((2, CHUNK, d), jnp.float32),
            pltpu.VMEM((2, CHUNK), jnp.int32),
            pltpu.VMEM((CHUNK, LANES), jnp.float32),
            pltpu.VMEM((b, d), jnp.float32),
            pltpu.VMEM((b, LANES), jnp.float32),
            pltpu.VMEM_SHARED((b, d), jnp.float32),
            pltpu.VMEM_SHARED((b, d), jnp.float32),
            pltpu.VMEM_SHARED((b, LANES), jnp.float32),
            pltpu.VMEM_SHARED((b, LANES), jnp.float32),
            pltpu.SemaphoreType.DMA,
            pltpu.SemaphoreType.DMA,
            pltpu.SemaphoreType.DMA,
            pltpu.SemaphoreType.DMA,
        ],
    )
    def sc_kernel(ref_hbm, alt_hbm, rid_hbm, aid_hbm, out_sums, out_cnts,
                  rows_v, idx_v, ones_v, zeros_v, zc_v,
                  acc_r, acc_a, cnt_r, cnt_a,
                  sem_r0, sem_r1, sem_i0, sem_i1):
        cid = lax.axis_index("c")
        sid = lax.axis_index("s")
        wid = cid * NS + sid
        base = wid * rows_per_w
        sems_r = (sem_r0, sem_r1)
        sems_i = (sem_i0, sem_i1)

        plan = [(src, ids, acc, cnt, base + j * CHUNK)
                for (src, ids, acc, cnt) in
                ((ref_hbm, rid_hbm, acc_r, cnt_r),
                 (alt_hbm, aid_hbm, acc_a, cnt_a))
                for j in range(m)]

        def start(i):
            slot = i % 2
            src, ids, _, _, r0 = plan[i]
            dr = pltpu.async_copy(src.at[pl.ds(r0, CHUNK)],
                                  rows_v.at[slot], sems_r[slot])
            di = pltpu.async_copy(ids.at[pl.ds(r0, CHUNK)],
                                  idx_v.at[slot], sems_i[slot])
            return dr, di

        pending = start(0)

        one_row = jnp.ones((LANES,), jnp.float32)
        for r in range(CHUNK):
            ones_v[r, :] = one_row

        @pl.when(sid == 0)
        def _():
            zrow = jnp.zeros((LANES,), jnp.float32)
            for r in range(b):
                for j in range(d // LANES):
                    zeros_v[r, pl.ds(j * LANES, LANES)] = zrow
                zc_v[r, :] = zrow
            pltpu.sync_copy(zeros_v, acc_r)
            pltpu.sync_copy(zeros_v, acc_a)
            pltpu.sync_copy(zc_v, cnt_r)
            pltpu.sync_copy(zc_v, cnt_a)

        plsc.subcore_barrier()

        for i in range(len(plan)):
            slot = i % 2
            nxt = start(i + 1) if i + 1 < len(plan) else None
            dr, di = pending
            dr.wait()
            di.wait()
            _, _, acc, cnt, _ = plan[i]
            pltpu.sync_copy(rows_v.at[slot], acc.at[idx_v.at[slot]], add=True)
            pltpu.sync_copy(ones_v, cnt.at[idx_v.at[slot]], add=True)
            pending = nxt

        plsc.subcore_barrier()

        @pl.when(sid == 0)
        def _():
            pltpu.sync_copy(acc_r, out_sums.at[cid, 0])
            pltpu.sync_copy(acc_a, out_sums.at[cid, 1])
            pltpu.sync_copy(cnt_r, out_cnts.at[cid, 0])
            pltpu.sync_copy(cnt_a, out_cnts.at[cid, 1])

    return sc_kernel


def _tc_partial_segsum(ref_flat, alt_flat, rid2, aid2, b, start_row):
    total, d = ref_flat.shape
    nb = (total - start_row) // BLK
    blk0 = start_row // BLK

    def body(r_ref, a_ref, rid_ref, aid_ref, sums_out, cnts_out, acc, cnt):
        i = pl.program_id(0)

        @pl.when(i == 0)
        def _():
            acc[...] = jnp.zeros_like(acc)
            cnt[...] = jnp.zeros_like(cnt)

        iota_b = lax.broadcasted_iota(jnp.int32, (b, BLK), 0)
        ohr = jnp.where(iota_b == rid_ref[...], 1.0, 0.0)
        oha = jnp.where(iota_b == aid_ref[...], 1.0, 0.0)
        acc[0] += jnp.dot(ohr, r_ref[...], preferred_element_type=jnp.float32)
        acc[1] += jnp.dot(oha, a_ref[...], preferred_element_type=jnp.float32)
        cnt[0] += jnp.sum(ohr, axis=1, keepdims=True)
        cnt[1] += jnp.sum(oha, axis=1, keepdims=True)

        @pl.when(i == nb - 1)
        def _():
            sums_out[...] = acc[...]
            cnts_out[...] = cnt[...]

    return pl.pallas_call(
        body,
        grid=(nb,),
        in_specs=[
            pl.BlockSpec((BLK, d), lambda i: (blk0 + i, 0)),
            pl.BlockSpec((BLK, d), lambda i: (blk0 + i, 0)),
            pl.BlockSpec((1, BLK), lambda i: (0, blk0 + i)),
            pl.BlockSpec((1, BLK), lambda i: (0, blk0 + i)),
        ],
        out_specs=[
            pl.BlockSpec((2, b, d), lambda i: (0, 0, 0)),
            pl.BlockSpec((2, b, 1), lambda i: (0, 0, 0)),
        ],
        out_shape=(
            jax.ShapeDtypeStruct((2, b, d), jnp.float32),
            jax.ShapeDtypeStruct((2, b, 1), jnp.float32),
        ),
        scratch_shapes=[
            pltpu.VMEM((2, b, d), jnp.float32),
            pltpu.VMEM((2, b, 1), jnp.float32),
        ],
    )(ref_flat, alt_flat, rid2, aid2)


def _tc_epilogue(sc_sums, sc_cnts, tc_sums, tc_cnts,
                 alt_W, ref_W, alt_b2, ref_b2, cw_vk, vt2):
    b = tc_sums.shape[1]
    k = alt_W.shape[0]
    v = cw_vk.shape[0]

    def body(ssum_ref, scnt_ref, tsum_ref, tcnt_ref, aW_ref, rW_ref,
             ab_ref, rb_ref, cw_ref, vt_ref, logits_out, ll_out):
        s = ssum_ref[...]
        c = scnt_ref[...]
        t = tsum_ref[...]
        tc = tcnt_ref[...]
        s_r = s[0, 0] + s[1, 0] + t[0]
        s_a = s[0, 1] + s[1, 1] + t[1]
        c_r = (c[0, 0] + c[1, 0])[:, 0:1] + tc[0]
        c_a = (c[0, 1] + c[1, 1])[:, 0:1] + tc[1]
        ll = (jnp.dot(s_a, aW_ref[...].T, preferred_element_type=jnp.float32)
              + jnp.dot(s_r, rW_ref[...].T, preferred_element_type=jnp.float32)
              + c_a * ab_ref[...] + c_r * rb_ref[...])
        cw = cw_ref[...]
        m = jnp.max(cw, axis=-1, keepdims=True)
        lw = cw - (m + jnp.log(jnp.sum(jnp.exp(cw - m), axis=-1,
                                       keepdims=True)))
        vt = vt_ref[...]
        iota_v = lax.broadcasted_iota(jnp.int32, (b, v), 1)
        oh = jnp.where(vt == iota_v, 1.0, 0.0)
        sel = jnp.dot(oh, lw, preferred_element_type=jnp.float32)
        tail = ll[:, 1:] + sel
        m2 = jnp.max(tail, axis=-1, keepdims=True)
        art = m2[:, 0] + jnp.log(jnp.sum(jnp.exp(tail - m2), axis=-1))
        logits_out[...] = (art - ll[:, 0]).reshape(1, b)
        ll_out[...] = jnp.concatenate([ll[:, 0:1], tail], axis=1)

    return pl.pallas_call(
        body,
        out_shape=(
            jax.ShapeDtypeStruct((1, b), jnp.float32),
            jax.ShapeDtypeStruct((b, k), jnp.float32),
        ),
    )(sc_sums, sc_cnts, tc_sums, tc_cnts,
      alt_W, ref_W, alt_b2, ref_b2, cw_vk, vt2)


def kernel(ref_flat, alt_flat, ref_seg_ids, alt_seg_ids, var_types_b,
           alt_W, alt_b, ref_W, ref_b, cluster_weights_pre_softmax_vk):
    total, d = ref_flat.shape
    k = alt_W.shape[0]
    b = var_types_b.shape[0]

    rid = ref_seg_ids.astype(jnp.int32)
    aid = alt_seg_ids.astype(jnp.int32)

    sc = _make_sc_segment_sum(d, b, SC_ROWS)
    sc_sums, sc_cnts = sc(ref_flat, alt_flat, rid, aid)

    tc_sums, tc_cnts = _tc_partial_segsum(
        ref_flat, alt_flat,
        rid.reshape(1, total),
        aid.reshape(1, total),
        b, SC_ROWS)

    logits2, ll = _tc_epilogue(
        sc_sums, sc_cnts, tc_sums, tc_cnts,
        alt_W, ref_W,
        alt_b.reshape(1, k), ref_b.reshape(1, k),
        cluster_weights_pre_softmax_vk,
        var_types_b.astype(jnp.int32).reshape(b, 1),
    )
    return (logits2.reshape(b), ll)


# speedup vs baseline: 1.1912x; 1.1912x over previous
import functools

import jax
import jax.numpy as jnp
from jax import lax
from jax.experimental import pallas as pl
from jax.experimental.pallas import tpu as pltpu
from jax.experimental.pallas import tpu_sc as plsc

NC = 2
NS = 16
LANES = 16
CHUNK = 128
SC_ROWS = 4096
BLK = 2048


def _make_sc_segment_sum(d, b, sc_rows):
    nw = NC * NS
    rows_per_w = sc_rows // nw
    m = rows_per_w // CHUNK
    mesh = plsc.VectorSubcoreMesh(
        core_axis_name="c", subcore_axis_name="s",
        num_cores=NC, num_subcores=NS)

    @functools.partial(
        pl.kernel,
        out_type=(
            jax.ShapeDtypeStruct((NC, 2, b, d), jnp.float32),
            jax.ShapeDtypeStruct((NC, 2, b, LANES), jnp.float32),
        ),
        mesh=mesh,
        scratch_types=[
            pltpu.VMEM((2, CHUNK, d), jnp.float32),
            pltpu.VMEM((2, CHUNK), jnp.int32),
            pltpu.VMEM((CHUNK, LANES), jnp.float32),
            pltpu.VMEM((b, d), jnp.float32),
            pltpu.VMEM((b, LANES), jnp.float32),
            pltpu.VMEM_SHARED((b, d), jnp.float32),
            pltpu.VMEM_SHARED((b, d), jnp.float32),
            pltpu.VMEM_SHARED((b, LANES), jnp.float32),
            pltpu.VMEM_SHARED((b, LANES), jnp.float32),
            pltpu.SemaphoreType.DMA,
            pltpu.SemaphoreType.DMA,
            pltpu.SemaphoreType.DMA,
            pltpu.SemaphoreType.DMA,
        ],
    )
    def sc_kernel(ref_hbm, alt_hbm, rid_hbm, aid_hbm, out_sums, out_cnts,
                  rows_v, idx_v, ones_v, zeros_v, zc_v,
                  acc_r, acc_a, cnt_r, cnt_a,
                  sem_r0, sem_r1, sem_i0, sem_i1):
        cid = lax.axis_index("c")
        sid = lax.axis_index("s")
        wid = cid * NS + sid
        base = wid * rows_per_w
        sems_r = (sem_r0, sem_r1)
        sems_i = (sem_i0, sem_i1)

        plan = [(src, ids, acc, cnt, base + j * CHUNK)
                for (src, ids, acc, cnt) in
                ((ref_hbm, rid_hbm, acc_r, cnt_r),
                 (alt_hbm, aid_hbm, acc_a, cnt_a))
                for j in range(m)]

        def start(i):
            slot = i % 2
            src, ids, _, _, r0 = plan[i]
            dr = pltpu.async_copy(src.at[pl.ds(r0, CHUNK)],
                                  rows_v.at[slot], sems_r[slot])
            di = pltpu.async_copy(ids.at[pl.ds(r0, CHUNK)],
                                  idx_v.at[slot], sems_i[slot])
            return dr, di

        pending = start(0)

        one_row = jnp.ones((LANES,), jnp.float32)
        for r in range(CHUNK):
            ones_v[r, :] = one_row

        @pl.when(sid == 0)
        def _():
            zrow = jnp.zeros((LANES,), jnp.float32)
            for r in range(b):
                for j in range(d // LANES):
                    zeros_v[r, pl.ds(j * LANES, LANES)] = zrow
                zc_v[r, :] = zrow
            pltpu.sync_copy(zeros_v, acc_r)
            pltpu.sync_copy(zeros_v, acc_a)
            pltpu.sync_copy(zc_v, cnt_r)
            pltpu.sync_copy(zc_v, cnt_a)

        plsc.subcore_barrier()

        for i in range(len(plan)):
            slot = i % 2
            nxt = start(i + 1) if i + 1 < len(plan) else None
            dr, di = pending
            dr.wait()
            di.wait()
            _, _, acc, cnt, _ = plan[i]
            pltpu.sync_copy(rows_v.at[slot], acc.at[idx_v.at[slot]], add=True)
            pltpu.sync_copy(ones_v, cnt.at[idx_v.at[slot]], add=True)
            pending = nxt

        plsc.subcore_barrier()

        @pl.when(sid == 0)
        def _():
            pltpu.sync_copy(acc_r, out_sums.at[cid, 0])
            pltpu.sync_copy(acc_a, out_sums.at[cid, 1])
            pltpu.sync_copy(cnt_r, out_cnts.at[cid, 0])
            pltpu.sync_copy(cnt_a, out_cnts.at[cid, 1])

    return sc_kernel


def _tc_partial_segsum(ref_flat, alt_flat, rid2, aid2, b, start_row):
    total, d = ref_flat.shape
    nb = (total - start_row) // BLK
    blk0 = start_row // BLK

    def body(r_ref, a_ref, rid_ref, aid_ref, sums_out, cnts_out, acc, cnt):
        i = pl.program_id(0)

        @pl.when(i == 0)
        def _():
            acc[...] = jnp.zeros_like(acc)
            cnt[...] = jnp.zeros_like(cnt)

        iota_b = lax.broadcasted_iota(jnp.int32, (b, BLK), 0)
        ohr = jnp.where(iota_b == rid_ref[...], 1.0, 0.0)
        oha = jnp.where(iota_b == aid_ref[...], 1.0, 0.0)
        acc[0] += jnp.dot(ohr, r_ref[...], preferred_element_type=jnp.float32)
        acc[1] += jnp.dot(oha, a_ref[...], preferred_element_type=jnp.float32)
        cnt[0] += jnp.sum(ohr, axis=1, keepdims=True)
        cnt[1] += jnp.sum(oha, axis=1, keepdims=True)

        @pl.when(i == nb - 1)
        def _():
            sums_out[...] = acc[...]
            cnts_out[...] = cnt[...]

    return pl.pallas_call(
        body,
        grid=(nb,),
        in_specs=[
            pl.BlockSpec((BLK, d), lambda i: (blk0 + i, 0)),
            pl.BlockSpec((BLK, d), lambda i: (blk0 + i, 0)),
            pl.BlockSpec((1, BLK), lambda i: (0, blk0 + i)),
            pl.BlockSpec((1, BLK), lambda i: (0, blk0 + i)),
        ],
        out_specs=[
            pl.BlockSpec((2, b, d), lambda i: (0, 0, 0)),
            pl.BlockSpec((2, b, 1), lambda i: (0, 0, 0)),
        ],
        out_shape=(
            jax.ShapeDtypeStruct((2, b, d), jnp.float32),
            jax.ShapeDtypeStruct((2, b, 1), jnp.float32),
        ),
        scratch_shapes=[
            pltpu.VMEM((2, b, d), jnp.float32),
            pltpu.VMEM((2, b, 1), jnp.float32),
        ],
    )(ref_flat, alt_flat, rid2, aid2)


def _tc_epilogue(sc_sums, sc_cnts, tc_sums, tc_cnts,
                 alt_W, ref_W, alt_b2, ref_b2, cw_vk, vt2):
    b = tc_sums.shape[1]
    k = alt_W.shape[0]
    v = cw_vk.shape[0]

    def body(ssum_ref, scnt_ref, tsum_ref, tcnt_ref, aW_ref, rW_ref,
             ab_ref, rb_ref, cw_ref, vt_ref, logits_out, ll_out):
        s = ssum_ref[...]
        c = scnt_ref[...]
        t = tsum_ref[...]
        tc = tcnt_ref[...]
        s_r = s[0, 0] + s[1, 0] + t[0]
        s_a = s[0, 1] + s[1, 1] + t[1]
        c_r = (c[0, 0] + c[1, 0])[:, 0:1] + tc[0]
        c_a = (c[0, 1] + c[1, 1])[:, 0:1] + tc[1]
        ll = (jnp.dot(s_a, aW_ref[...].T, preferred_element_type=jnp.float32)
              + jnp.dot(s_r, rW_ref[...].T, preferred_element_type=jnp.float32)
              + c_a * ab_ref[...] + c_r * rb_ref[...])
        cw = cw_ref[...]
        m = jnp.max(cw, axis=-1, keepdims=True)
        lw = cw - (m + jnp.log(jnp.sum(jnp.exp(cw - m), axis=-1,
                                       keepdims=True)))
        vt = vt_ref[...]
        iota_v = lax.broadcasted_iota(jnp.int32, (b, v), 1)
        oh = jnp.where(vt == iota_v, 1.0, 0.0)
        sel = jnp.dot(oh, lw, preferred_element_type=jnp.float32)
        tail = ll[:, 1:] + sel
        m2 = jnp.max(tail, axis=-1, keepdims=True)
        art = m2[:, 0] + jnp.log(jnp.sum(jnp.exp(tail - m2), axis=-1))
        logits_out[...] = (art - ll[:, 0]).reshape(1, b)
        ll_out[...] = jnp.concatenate([ll[:, 0:1], tail], axis=1)

    return pl.pallas_call(
        body,
        out_shape=(
            jax.ShapeDtypeStruct((1, b), jnp.float32),
            jax.ShapeDtypeStruct((b, k), jnp.float32),
        ),
    )(sc_sums, sc_cnts, tc_sums, tc_cnts,
      alt_W, ref_W, alt_b2, ref_b2, cw_vk, vt2)


def kernel(ref_flat, alt_flat, ref_seg_ids, alt_seg_ids, var_types_b,
           alt_W, alt_b, ref_W, ref_b, cluster_weights_pre_softmax_vk):
    total, d = ref_flat.shape
    k = alt_W.shape[0]
    b = var_types_b.shape[0]

    rid = ref_seg_ids.astype(jnp.int32)
    aid = alt_seg_ids.astype(jnp.int32)

    sc = _make_sc_segment_sum(d, b, SC_ROWS)
    sc_sums, sc_cnts = sc(ref_flat, alt_flat, rid, aid)

    tc_sums, tc_cnts = _tc_partial_segsum(
        ref_flat, alt_flat,
        rid.reshape(1, total),
        aid.reshape(1, total),
        b, SC_ROWS)

    logits2, ll = _tc_epilogue(
        sc_sums, sc_cnts, tc_sums, tc_cnts,
        alt_W, ref_W,
        alt_b.reshape(1, k), ref_b.reshape(1, k),
        cluster_weights_pre_softmax_vk,
        var_types_b.astype(jnp.int32).reshape(b, 1),
    )
    return (logits2.reshape(b), ll)
